# baseline (speedup 1.0000x reference)
import functools

import jax
import jax.numpy as jnp
from jax import lax
from jax.experimental import pallas as pl
from jax.experimental.pallas import tpu as pltpu
from jax.experimental.pallas import tpu_sc as plsc

_BINS = 100
_HMIN = 60.0
_HMAX = 120.0

_NC = 2
_NS = 16
_NW = _NC * _NS
_L = 16

_CE = 2048
_DT_OFF = 128

_C2 = 1.0 / 24.0
_C3 = 1.0 / 720.0
_C4 = 1.0 / 40320.0
_C5 = 1.0 / 3628800.0


def _cos_poly(x):
    u = x * x
    return ((((-_C5 * u + _C4) * u - _C3) * u + _C2) * u - 0.5) * u + 1.0


def _cosh_poly(x):
    u = x * x
    return ((((_C5 * u + _C4) * u + _C3) * u + _C2) * u + 0.5) * u + 1.0


def _sqrt_sc(x):
    i = lax.bitcast_convert_type(x, jnp.int32)
    i = 0x5F3759DF - lax.shift_right_logical(i, 1)
    y = lax.bitcast_convert_type(i, jnp.float32)
    xh = 0.5 * x
    y = y * (1.5 - xh * y * y)
    y = y * (1.5 - xh * y * y)
    y = y * (1.5 - xh * y * y)
    return x * y


def _bin_index(x):
    t = (x - _HMIN) * _BINS / (_HMAX - _HMIN)
    i0 = jnp.minimum(jnp.maximum(t.astype(jnp.int32), 0), _BINS - 1)
    valid = (x >= _HMIN) & (x <= _HMAX)
    return jnp.where(valid, i0, _BINS)


def _make_sc_body(n):
    nch = n // _CE
    kmax = -(-nch // _NW)
    tail = n - nch * _CE
    tvec = tail // 16
    trem = tail - tvec * 16

    def body(corr_hbm, mc_hbm, dt_hbm, out_hbm, bc, bm, bd, hist):
        wid = lax.axis_index("s") * _NC + lax.axis_index("c")

        lane = lax.broadcasted_iota(jnp.int32, (_L,), 0)
        ones = jnp.full((_L,), 1.0, jnp.float32)
        zeros = jnp.zeros((_L,), jnp.float32)
        col = [jnp.full((_L,), c, jnp.int32) for c in range(9)]
        mc_base = lane * 256
        dt_base = lane * 256 + _DT_OFF

        for k in range(_L * 256 // _L):
            hist[pl.ds(k * _L, _L)] = zeros

        def do_vec(j, mask):
            rv = lane + 16 * j
            g = lambda ref, c: plsc.load_gather(ref, [rv, col[c]], mask=mask)
            c0 = g(bc, 0)
            c1 = g(bc, 1)
            m0 = g(bm, 0)
            m1 = g(bm, 1)
            f1 = g(bm, 4)
            f2 = g(bm, 5)
            e1 = g(bm, 6)
            e2 = g(bm, 7)
            xd = g(bd, 8)

            q = (c0 * c1) * (m0 * m1)
            ch = _cosh_poly(e1 - e2)
            co = _cos_poly(f1 - f2)
            mz2 = 2.0 * q * (ch - co)
            mz = _sqrt_sc(jnp.maximum(mz2, 1e-30))

            imc = _bin_index(mz)
            idt = _bin_index(xd)
            plsc.addupdate_scatter(hist, [mc_base + imc], ones, mask=mask)
            plsc.addupdate_scatter(hist, [dt_base + idt], ones, mask=mask)

        def do_chunk(ebase, nvec, ne):
            off = pl.multiple_of(ebase, 8)
            pltpu.sync_copy(corr_hbm.at[pl.ds(off, ne)], bc.at[pl.ds(0, ne)])
            pltpu.sync_copy(mc_hbm.at[pl.ds(off, ne)], bm.at[pl.ds(0, ne)])
            pltpu.sync_copy(dt_hbm.at[pl.ds(off, ne)], bd.at[pl.ds(0, ne)])

            @plsc.parallel_loop(0, nvec, 1, unroll=8)
            def _vec_body(j):
                do_vec(j, None)

        def chunk_body(k, carry):
            c = k * _NW + wid

            @pl.when(c < nch)
            def _():
                do_chunk(c * _CE, _CE // 16, _CE)

            return carry

        lax.fori_loop(0, kmax, chunk_body, 0)

        if tail:
            @pl.when(wid == 0)
            def _():
                do_chunk(nch * _CE, tvec, tail)
                if trem:
                    do_vec(tvec, lane < trem)

        pltpu.sync_copy(hist, out_hbm.at[wid])

    return body


def _kl_body(h_ref, out_ref):
    s = jnp.sum(h_ref[...], axis=0, keepdims=True)
    hm = s[:, 0:_BINS]
    hd = s[:, _DT_OFF:_DT_OFF + _BINS]
    pw = jnp.where(hd > 0.0, hd * (jnp.log(jnp.where(hd > 0.0, hd, 1.0)) - hm), 0.0)
    out_ref[...] = (jnp.sum(pw) / float(_BINS)).reshape(1, 1)


def kernel(corr, mc, dt):
    n = corr.shape[0]

    sc_hist = functools.partial(
        pl.kernel,
        mesh=plsc.VectorSubcoreMesh(core_axis_name="c", subcore_axis_name="s"),
        out_type=jax.ShapeDtypeStruct((_NW, _L * 256), jnp.float32),
        scratch_types=[
            pltpu.VMEM((_CE, 2), jnp.float32),
            pltpu.VMEM((_CE, 8), jnp.float32),
            pltpu.VMEM((_CE, 9), jnp.float32),
            pltpu.VMEM((_L * 256,), jnp.float32),
        ],
        compiler_params=pltpu.CompilerParams(
            needs_layout_passes=False, use_tc_tiling_on_sc=False),
    )(_make_sc_body(n))
    hists = sc_hist(corr, mc, dt)

    out = pl.pallas_call(
        _kl_body,
        in_specs=[pl.BlockSpec((_NW * _L, 256), lambda: (0, 0))],
        out_specs=pl.BlockSpec((1, 1), lambda: (0, 0)),
        out_shape=jax.ShapeDtypeStruct((1, 1), jnp.float32),
    )(hists.reshape(_NW * _L, 256))
    return out[0, 0]


# speedup vs baseline: 21.0230x; 21.0230x over previous
import functools

import jax
import jax.numpy as jnp
from jax import lax
from jax.experimental import pallas as pl
from jax.experimental.pallas import tpu as pltpu
from jax.experimental.pallas import tpu_sc as plsc

_BINS = 100
_HMIN = 60.0
_HMAX = 120.0

_BN = 131072
_NC = 2
_NS = 16
_NW = _NC * _NS
_L = 16

_CE = 16384
_DT_OFF = 128


def _bin_index(x):
    t = (x - _HMIN) * _BINS / (_HMAX - _HMIN)
    i0 = jnp.clip(jnp.floor(t).astype(jnp.int32), 0, _BINS - 1)
    valid = (x >= _HMIN) & (x <= _HMAX)
    return jnp.where(valid, i0, _BINS)


def _idx_body(corr_ref, mc_ref, dtc_ref, imc_ref, idt_ref):
    c0 = corr_ref[0]
    c1 = corr_ref[1]
    m0 = mc_ref[0]
    m1 = mc_ref[1]
    f1 = mc_ref[4]
    f2 = mc_ref[5]
    e1 = mc_ref[6]
    e2 = mc_ref[7]
    x_dt = dtc_ref[0, 0]

    q = (c0 * c1) * (m0 * m1)
    de = e1 - e2
    cosh_de = 0.5 * (jnp.exp(de) + jnp.exp(-de))
    mz2 = 2.0 * q * (cosh_de - jnp.cos(f1 - f2))
    mz = jnp.sqrt(jnp.maximum(mz2, 0.0))

    imc_ref[...] = _bin_index(mz)
    idt_ref[...] = _bin_index(x_dt)


def _make_sc_body(n):
    nch = n // _CE
    kmax = -(-nch // _NW)
    tail = n - nch * _CE
    tvec = tail // 16
    trem = tail - tvec * 16

    def body(imc_hbm, idt_hbm, out_hbm, buf, hist):
        wid = lax.axis_index("s") * _NC + lax.axis_index("c")

        lane = lax.broadcasted_iota(jnp.int32, (_L,), 0)
        ones = jnp.full((_L,), 1.0, jnp.float32)
        zeros = jnp.zeros((_L,), jnp.float32)
        mc_base = lane * 256
        dt_base = lane * 256 + _DT_OFF

        for k in range(256):
            hist[pl.ds(k * _L, _L)] = zeros

        def scan_chunk(src_hbm, lane_base, off, nvec, nw, rem=0):
            pltpu.sync_copy(src_hbm.at[pl.ds(off, nw)], buf.at[pl.ds(0, nw)])

            @plsc.parallel_loop(0, nvec, 1, unroll=8)
            def _vec(j):
                v = buf[pl.ds(j * _L, _L)]
                plsc.addupdate_scatter(hist, [lane_base + v], ones)

            if rem:
                mask = lane < rem
                v = plsc.load_gather(buf, [lane + nvec * _L], mask=mask)
                plsc.addupdate_scatter(hist, [lane_base + v], ones, mask=mask)

        def chunk_body(k, carry):
            c = k * _NW + wid

            @pl.when(c < nch)
            def _():
                off = pl.multiple_of(c * _CE, 8)
                scan_chunk(imc_hbm, mc_base, off, _CE // 16, _CE)
                scan_chunk(idt_hbm, dt_base, off, _CE // 16, _CE)

            return carry

        lax.fori_loop(0, kmax, chunk_body, 0)

        if tail:
            @pl.when(wid == 0)
            def _():
                off = pl.multiple_of(nch * _CE, 8)
                scan_chunk(imc_hbm, mc_base, off, tvec, tail, trem)
                scan_chunk(idt_hbm, dt_base, off, tvec, tail, trem)

        pltpu.sync_copy(hist, out_hbm.at[wid])

    return body


def _kl_body(h_ref, out_ref):
    s = jnp.sum(h_ref[...], axis=0, keepdims=True)
    hm = s[:, 0:_BINS]
    hd = s[:, _DT_OFF:_DT_OFF + _BINS]
    pw = jnp.where(hd > 0.0, hd * (jnp.log(jnp.where(hd > 0.0, hd, 1.0)) - hm), 0.0)
    out_ref[...] = (jnp.sum(pw) / float(_BINS)).reshape(1, 1)


def kernel(corr, mc, dt):
    n = corr.shape[0]
    nblk = -(-n // _BN)

    corr_t = corr.T
    mc_t = mc.T
    dt_t = dt.T

    imc, idt = pl.pallas_call(
        _idx_body,
        grid=(nblk,),
        in_specs=[
            pl.BlockSpec((2, _BN), lambda i: (0, i)),
            pl.BlockSpec((8, _BN), lambda i: (0, i)),
            pl.BlockSpec((1, 1, _BN), lambda i: (8, 0, i)),
        ],
        out_specs=[
            pl.BlockSpec((_BN,), lambda i: (i,)),
            pl.BlockSpec((_BN,), lambda i: (i,)),
        ],
        out_shape=[
            jax.ShapeDtypeStruct((n,), jnp.int32),
            jax.ShapeDtypeStruct((n,), jnp.int32),
        ],
    )(corr_t, mc_t, dt_t.reshape(9, 1, n))

    sc_hist = functools.partial(
        pl.kernel,
        mesh=plsc.VectorSubcoreMesh(core_axis_name="c", subcore_axis_name="s"),
        out_type=jax.ShapeDtypeStruct((_NW, _L * 256), jnp.float32),
        scratch_types=[
            pltpu.VMEM((_CE,), jnp.int32),
            pltpu.VMEM((_L * 256,), jnp.float32),
        ],
        compiler_params=pltpu.CompilerParams(needs_layout_passes=False),
    )(_make_sc_body(n))
    hists = sc_hist(imc, idt)

    out = pl.pallas_call(
        _kl_body,
        in_specs=[pl.BlockSpec((_NW * _L, 256), lambda: (0, 0))],
        out_specs=pl.BlockSpec((1, 1), lambda: (0, 0)),
        out_shape=jax.ShapeDtypeStruct((1, 1), jnp.float32),
    )(hists.reshape(_NW * _L, 256))
    return out[0, 0]
